# Initial kernel scaffold; baseline (speedup 1.0000x reference)
import jax
import jax.numpy as jnp
from jax.experimental import pallas as pl


def kernel(s, v, pos, edge_index, phi_w1, phi_b1, phi_w2, phi_b2, Ws_w, Ws_b, Wv_w, Wv_b, Us_w1, Us_b1, Us_w2, Us_b2):
    raise NotImplementedError("write your pallas kernel here")


# speedup vs baseline: 20.0294x; 20.0294x over previous
import functools

import jax
import jax.numpy as jnp
from jax import lax
from jax.experimental import pallas as pl
from jax.experimental.pallas import tpu as pltpu
from jax.experimental.pallas import tpu_sc as plsc

N_RBF = 20
CUTOFF = 5.0
H = 128

_NC, _NS = 2, 16
_NW = _NC * _NS
_EG = 128


def _k1_body(s_ref, v0_ref, v1_ref, v2_ref, WsT_ref, Wsb_ref, WvT_ref,
             o0_ref, o1_ref, o2_ref, o3_ref):
    WvT = WvT_ref[...]
    o0_ref[...] = jnp.dot(s_ref[...], WsT_ref[...],
                          preferred_element_type=jnp.float32) + Wsb_ref[...]
    o1_ref[...] = jnp.dot(v0_ref[...], WvT, preferred_element_type=jnp.float32)
    o2_ref[...] = jnp.dot(v1_ref[...], WvT, preferred_element_type=jnp.float32)
    o3_ref[...] = jnp.dot(v2_ref[...], WvT, preferred_element_type=jnp.float32)


def _node_tables(s, v, Ws_w, Ws_b, Wv_w, block_n=2000):
    n = s.shape[0]
    grid = n // block_n
    bs = pl.BlockSpec((block_n, H), lambda i: (i, 0))
    ws = pl.BlockSpec((H, H), lambda i: (0, 0))
    return pl.pallas_call(
        _k1_body,
        grid=(grid,),
        in_specs=[bs, bs, bs, bs, ws, pl.BlockSpec((1, H), lambda i: (0, 0)), ws],
        out_specs=[bs, bs, bs, bs],
        out_shape=[jax.ShapeDtypeStruct((n, H), jnp.float32)] * 4,
    )(s, v[:, 0, :], v[:, 1, :], v[:, 2, :], Ws_w.T, Ws_b.reshape(1, H), Wv_w.T)


def _sc_pos(posx, posy, posz, idx_i, idx_j):
    n = posx.shape[0]
    e = idx_i.shape[0]
    ngroups = e // _EG
    mesh = plsc.VectorSubcoreMesh(core_axis_name="c", subcore_axis_name="s")
    out3d = jax.ShapeDtypeStruct((ngroups, 8, _EG), jnp.float32)

    @functools.partial(
        pl.kernel,
        out_type=[out3d, out3d, out3d, out3d],
        mesh=mesh,
        compiler_params=pltpu.CompilerParams(needs_layout_passes=False),
        scratch_types=[
            pltpu.VMEM((n,), jnp.float32),
            pltpu.VMEM((n,), jnp.float32),
            pltpu.VMEM((n,), jnp.float32),
            pltpu.VMEM((_EG,), jnp.int32),
            pltpu.VMEM((_EG,), jnp.int32),
            pltpu.VMEM((8, _EG), jnp.float32),
            pltpu.VMEM((8, _EG), jnp.float32),
            pltpu.VMEM((8, _EG), jnp.float32),
            pltpu.VMEM((8, _EG), jnp.float32),
        ],
    )
    def k(px_hbm, py_hbm, pz_hbm, ii_hbm, jj_hbm,
          rx_hbm, ry_hbm, rz_hbm, d2_hbm,
          px_v, py_v, pz_v, ii_v, jj_v, sx, sy, sz, sd):
        c = lax.axis_index("c")
        s = lax.axis_index("s")
        w = c * _NS + s
        pltpu.sync_copy(px_hbm, px_v)
        pltpu.sync_copy(py_hbm, py_v)
        pltpu.sync_copy(pz_hbm, pz_v)
        nt = (ngroups - w + _NW - 1) // _NW

        def body(t, carry):
            g = w + t * _NW
            off = pl.multiple_of(g * _EG, _EG)
            pltpu.sync_copy(ii_hbm.at[pl.ds(off, _EG)], ii_v)
            pltpu.sync_copy(jj_hbm.at[pl.ds(off, _EG)], jj_v)
            for sub in range(8):
                i16 = ii_v[pl.ds(sub * 16, 16)]
                j16 = jj_v[pl.ds(sub * 16, 16)]
                rx = plsc.load_gather(px_v, [j16]) - plsc.load_gather(px_v, [i16])
                ry = plsc.load_gather(py_v, [j16]) - plsc.load_gather(py_v, [i16])
                rz = plsc.load_gather(pz_v, [j16]) - plsc.load_gather(pz_v, [i16])
                d2 = rx * rx + ry * ry + rz * rz
                for r in range(8):
                    sx[r, pl.ds(sub * 16, 16)] = rx
                    sy[r, pl.ds(sub * 16, 16)] = ry
                    sz[r, pl.ds(sub * 16, 16)] = rz
                    sd[r, pl.ds(sub * 16, 16)] = d2
            pltpu.sync_copy(sx, rx_hbm.at[g])
            pltpu.sync_copy(sy, ry_hbm.at[g])
            pltpu.sync_copy(sz, rz_hbm.at[g])
            pltpu.sync_copy(sd, d2_hbm.at[g])
            return carry

        lax.fori_loop(0, nt, body, 0)

    return k(posx, posy, posz, idx_i, idx_j)


def _sc_gather(t0, t1, t2, t3, idx_j):
    e = idx_j.shape[0]
    ngroups = e // _EG
    mesh = plsc.VectorSubcoreMesh(core_axis_name="c", subcore_axis_name="s")
    out2d = jax.ShapeDtypeStruct((e, H), jnp.float32)

    @functools.partial(
        pl.kernel,
        out_type=[out2d, out2d, out2d, out2d],
        mesh=mesh,
        compiler_params=pltpu.CompilerParams(needs_layout_passes=False),
        scratch_types=[
            pltpu.VMEM((_EG,), jnp.int32),
            pltpu.VMEM((_EG, H), jnp.float32),
            pltpu.VMEM((_EG, H), jnp.float32),
            pltpu.VMEM((_EG, H), jnp.float32),
            pltpu.VMEM((_EG, H), jnp.float32),
            pltpu.SemaphoreType.DMA,
            pltpu.SemaphoreType.DMA,
            pltpu.SemaphoreType.DMA,
            pltpu.SemaphoreType.DMA,
        ],
    )
    def k(t0_hbm, t1_hbm, t2_hbm, t3_hbm, jj_hbm,
          o0_hbm, o1_hbm, o2_hbm, o3_hbm,
          jj_v, b0, b1, b2, b3, s0, s1, s2, s3):
        c = lax.axis_index("c")
        s = lax.axis_index("s")
        w = c * _NS + s
        nt = (ngroups - w + _NW - 1) // _NW
        tabs = (t0_hbm, t1_hbm, t2_hbm, t3_hbm)
        outs = (o0_hbm, o1_hbm, o2_hbm, o3_hbm)
        bufs = (b0, b1, b2, b3)
        sems = (s0, s1, s2, s3)

        def body(t, carry):
            g = w + t * _NW
            off = pl.multiple_of(g * _EG, _EG)
            pltpu.sync_copy(jj_hbm.at[pl.ds(off, _EG)], jj_v)
            cps = [pltpu.async_copy(tb.at[jj_v], bb, sm)
                   for tb, bb, sm in zip(tabs, bufs, sems)]
            for cp, bb, oo in zip(cps, bufs, outs):
                cp.wait()
                pltpu.sync_copy(bb, oo.at[pl.ds(off, _EG), :])
            return carry

        lax.fori_loop(0, nt, body, 0)

    return k(t0, t1, t2, t3, idx_j)


_BB = 10
_BE = _BB * _EG


def _k3_body(fs_ref, f0_ref, f1_ref, f2_ref, rx_ref, ry_ref, rz_ref, d2_ref,
             phi1pT_ref, b1c_ref, phi2_ref, b2c_ref, Wvb_ref,
             ms_ref, m0_ref, m1_ref, m2_ref):
    d2 = d2_ref[...]
    dist = jnp.sqrt(d2)
    invu = 1.0 / (dist + 1e-09)
    cv = 0.5 * (jnp.cos((jnp.pi / CUTOFF) * dist) + 1.0) \
        * (dist < CUTOFF).astype(jnp.float32)
    scl = cv / dist
    dist4 = jnp.concatenate([dist, dist, dist, dist], axis=1)
    scl4 = jnp.concatenate([scl, scl, scl, scl], axis=1)
    kidx = jax.lax.broadcasted_iota(jnp.int32, (1, 32, 1), 1)
    freq = jnp.where(kidx < N_RBF,
                     (kidx.astype(jnp.float32) + 1.0) * (jnp.pi / CUTOFF),
                     0.0)
    rbfT = jnp.sin(dist4 * freq) * scl4

    phi1pT = phi1pT_ref[...]
    phi2 = phi2_ref[...]
    b1c = b1c_ref[...]
    b2c = b2c_ref[...]
    w_rows, ux_rows, uy_rows, uz_rows = [], [], [], []
    for g in range(_BB):
        h1 = jnp.dot(phi1pT, rbfT[g],
                     preferred_element_type=jnp.float32) + b1c
        h = h1 * jax.nn.sigmoid(h1)
        wT = jnp.dot(phi2, h, preferred_element_type=jnp.float32) + b2c
        w_rows.append(wT.T)
        iv = invu[g]
        ux_rows.append((rx_ref[g] * iv).T[:, 0:1])
        uy_rows.append((ry_ref[g] * iv).T[:, 0:1])
        uz_rows.append((rz_ref[g] * iv).T[:, 0:1])
    w = jnp.concatenate(w_rows, axis=0)
    ux = jnp.concatenate(ux_rows, axis=0)
    uy = jnp.concatenate(uy_rows, axis=0)
    uz = jnp.concatenate(uz_rows, axis=0)

    ms_ref[...] = fs_ref[...] * w
    proj = ux * f0_ref[...] + uy * f1_ref[...] + uz * f2_ref[...]
    qv = (proj + Wvb_ref[...]) * w
    m0_ref[...] = ux * qv
    m1_ref[...] = uy * qv
    m2_ref[...] = uz * qv


def _edge_compute(fS, f0, f1, f2, r8x, r8y, r8z, d28,
                  phi_w1, phi_b1, phi_w2, phi_b2, Wv_b):
    e = fS.shape[0]
    grid = e // _BE
    phi1pT = jnp.zeros((H, 32), jnp.float32).at[:, :N_RBF].set(phi_w1)
    ebs = pl.BlockSpec((_BE, H), lambda i: (i, 0))
    rbs = pl.BlockSpec((_BB, 8, _EG), lambda i: (i, 0, 0))
    return pl.pallas_call(
        _k3_body,
        grid=(grid,),
        in_specs=[
            ebs, ebs, ebs, ebs, rbs, rbs, rbs, rbs,
            pl.BlockSpec((H, 32), lambda i: (0, 0)),
            pl.BlockSpec((H, 1), lambda i: (0, 0)),
            pl.BlockSpec((H, H), lambda i: (0, 0)),
            pl.BlockSpec((H, 1), lambda i: (0, 0)),
            pl.BlockSpec((1, H), lambda i: (0, 0)),
        ],
        out_specs=[ebs, ebs, ebs, ebs],
        out_shape=[jax.ShapeDtypeStruct((e, H), jnp.float32)] * 4,
    )(fS, f0, f1, f2, r8x, r8y, r8z, d28,
      phi1pT, phi_b1.reshape(H, 1), phi_w2, phi_b2.reshape(H, 1),
      Wv_b.reshape(1, H))


def _sc_scatter(ms, mv0, mv1, mv2, idx_i, n):
    e = idx_i.shape[0]
    ngroups = e // _EG
    rows0 = 624
    tail = n - rows0 * _NS
    mesh = plsc.VectorSubcoreMesh(core_axis_name="c", subcore_axis_name="s")
    outn = jax.ShapeDtypeStruct((n, H), jnp.float32)

    @functools.partial(
        pl.kernel,
        out_type=[outn, outn, outn, outn],
        mesh=mesh,
        compiler_params=pltpu.CompilerParams(needs_layout_passes=False),
        scratch_types=[
            pltpu.VMEM((_EG,), jnp.int32),
            pltpu.VMEM((_EG, H), jnp.float32),
            pltpu.VMEM((16, H), jnp.float32),
            pltpu.VMEM_SHARED((n, H), jnp.float32),
        ],
    )
    def k(d0_hbm, d1_hbm, d2_hbm, d3_hbm, ii_hbm,
          o0_hbm, o1_hbm, o2_hbm, o3_hbm,
          ii_v, buf, zbuf, acc):
        c = lax.axis_index("c")
        s = lax.axis_index("s")

        def zfill(t, carry):
            r = t // 8
            kk = (t % 8) * 16
            zbuf[r, pl.ds(kk, 16)] = jnp.zeros((16,), jnp.float32)
            return carry

        lax.fori_loop(0, 16 * 8, zfill, 0)

        def do_comp(dat_hbm, out_hbm):
            base = pl.multiple_of(s * rows0, 8)

            def zero_body(t, carry):
                zoff = pl.multiple_of(base + t * 16, 8)
                pltpu.sync_copy(zbuf, acc.at[pl.ds(zoff, 16), :])
                return carry

            lax.fori_loop(0, rows0 // 16, zero_body, 0)

            @pl.when(s == _NS - 1)
            def _():
                pltpu.sync_copy(zbuf, acc.at[pl.ds(rows0 * _NS, tail), :])

            plsc.subcore_barrier()
            nt = (ngroups - s + _NS - 1) // _NS

            def body(t, carry):
                g = s + t * _NS
                off = pl.multiple_of(g * _EG, _EG)
                pltpu.sync_copy(ii_hbm.at[pl.ds(off, _EG)], ii_v)
                pltpu.sync_copy(dat_hbm.at[pl.ds(off, _EG), :], buf)
                pltpu.sync_copy(buf, acc.at[ii_v], add=True)
                return carry

            lax.fori_loop(0, nt, body, 0)
            plsc.subcore_barrier()
            pltpu.sync_copy(acc.at[pl.ds(base, rows0), :],
                            out_hbm.at[pl.ds(base, rows0), :])

            @pl.when(s == _NS - 1)
            def _():
                pltpu.sync_copy(acc.at[pl.ds(rows0 * _NS, tail), :],
                                out_hbm.at[pl.ds(rows0 * _NS, tail), :])

        pairs = ((d0_hbm, o0_hbm), (d1_hbm, o1_hbm),
                 (d2_hbm, o2_hbm), (d3_hbm, o3_hbm))
        for ci, (dat, out) in enumerate(pairs):
            @pl.when(c == ci // 2)
            def _():
                do_comp(dat, out)

    return k(ms, mv0, mv1, mv2, idx_i)


def _k5_body(s_ref, v_ref, ms_ref, m0_ref, m1_ref, m2_ref,
             Us1T_ref, b1_ref, Us2T_ref, b2_ref, sout_ref, vout_ref):
    h1 = jnp.dot(ms_ref[...], Us1T_ref[...],
                 preferred_element_type=jnp.float32) + b1_ref[...]
    h = h1 * jax.nn.sigmoid(h1)
    ds = jnp.dot(h, Us2T_ref[...], preferred_element_type=jnp.float32) + b2_ref[...]
    sout_ref[...] = s_ref[...] + ds
    vout_ref[:, 0, :] = v_ref[:, 0, :] + m0_ref[...]
    vout_ref[:, 1, :] = v_ref[:, 1, :] + m1_ref[...]
    vout_ref[:, 2, :] = v_ref[:, 2, :] + m2_ref[...]


def _node_update(s, v, ms, m0, m1, m2, Us_w1, Us_b1, Us_w2, Us_b2,
                 block_n=2000):
    n = s.shape[0]
    grid = n // block_n
    bs = pl.BlockSpec((block_n, H), lambda i: (i, 0))
    vs = pl.BlockSpec((block_n, 3, H), lambda i: (i, 0, 0))
    ws = pl.BlockSpec((H, H), lambda i: (0, 0))
    cs = pl.BlockSpec((1, H), lambda i: (0, 0))
    return pl.pallas_call(
        _k5_body,
        grid=(grid,),
        in_specs=[bs, vs, bs, bs, bs, bs, ws, cs, ws, cs],
        out_specs=[bs, vs],
        out_shape=[
            jax.ShapeDtypeStruct((n, H), jnp.float32),
            jax.ShapeDtypeStruct((n, 3, H), jnp.float32),
        ],
    )(s, v, ms, m0, m1, m2, Us_w1.T, Us_b1.reshape(1, H),
      Us_w2.T, Us_b2.reshape(1, H))


def kernel(s, v, pos, edge_index, phi_w1, phi_b1, phi_w2, phi_b2, Ws_w, Ws_b,
           Wv_w, Wv_b, Us_w1, Us_b1, Us_w2, Us_b2):
    n = s.shape[0]
    i = edge_index[0]
    j = edge_index[1]
    posx = pos[:, 0]
    posy = pos[:, 1]
    posz = pos[:, 2]

    t0, t1, t2, t3 = _node_tables(s, v, Ws_w, Ws_b, Wv_w)
    r8x, r8y, r8z, d28 = _sc_pos(posx, posy, posz, i, j)
    fS, f0, f1, f2 = _sc_gather(t0, t1, t2, t3, j)
    ms_e, m0_e, m1_e, m2_e = _edge_compute(
        fS, f0, f1, f2, r8x, r8y, r8z, d28,
        phi_w1, phi_b1, phi_w2, phi_b2, Wv_b)
    aS, a0, a1, a2 = _sc_scatter(ms_e, m0_e, m1_e, m2_e, i, n)
    s_out, v_out = _node_update(s, v, aS, a0, a1, a2,
                                Us_w1, Us_b1, Us_w2, Us_b2)
    return (s_out, v_out)


# speedup vs baseline: 35.7122x; 1.7830x over previous
import functools

import jax
import jax.numpy as jnp
from jax import lax
from jax.experimental import pallas as pl
from jax.experimental.pallas import tpu as pltpu
from jax.experimental.pallas import tpu_sc as plsc

N_RBF = 20
CUTOFF = 5.0
H = 128

_NC, _NS = 2, 16
_NW = _NC * _NS
_EG = 128


def _pack_pair(a, b):
    a_u = jax.lax.bitcast_convert_type(
        a.astype(jnp.bfloat16).astype(jnp.float32), jnp.uint32)
    b_u = jax.lax.bitcast_convert_type(
        b.astype(jnp.bfloat16).astype(jnp.float32), jnp.uint32)
    word = (a_u >> 16) | (b_u & jnp.uint32(0xFFFF0000))
    return jax.lax.bitcast_convert_type(word, jnp.float32)


def _unpack_lo(w):
    u = jax.lax.bitcast_convert_type(w, jnp.uint32)
    return jax.lax.bitcast_convert_type(u << 16, jnp.float32)


def _unpack_hi(w):
    u = jax.lax.bitcast_convert_type(w, jnp.uint32)
    return jax.lax.bitcast_convert_type(u & jnp.uint32(0xFFFF0000), jnp.float32)


def _k1_body(s_ref, v0_ref, v1_ref, v2_ref, WsT_ref, Wsb_ref, WvT_ref,
             o0_ref, o1_ref):
    WvT = WvT_ref[...]
    sW = jnp.dot(s_ref[...], WsT_ref[...],
                 preferred_element_type=jnp.float32) + Wsb_ref[...]
    vW0 = jnp.dot(v0_ref[...], WvT, preferred_element_type=jnp.float32)
    vW1 = jnp.dot(v1_ref[...], WvT, preferred_element_type=jnp.float32)
    vW2 = jnp.dot(v2_ref[...], WvT, preferred_element_type=jnp.float32)
    o0_ref[...] = _pack_pair(sW, vW0)
    o1_ref[...] = _pack_pair(vW1, vW2)


def _node_tables(s, v, Ws_w, Ws_b, Wv_w, block_n=2000):
    n = s.shape[0]
    grid = n // block_n
    bs = pl.BlockSpec((block_n, H), lambda i: (i, 0))
    ws = pl.BlockSpec((H, H), lambda i: (0, 0))
    return pl.pallas_call(
        _k1_body,
        grid=(grid,),
        in_specs=[bs, bs, bs, bs, ws, pl.BlockSpec((1, H), lambda i: (0, 0)), ws],
        out_specs=[bs, bs],
        out_shape=[jax.ShapeDtypeStruct((n, H), jnp.float32)] * 2,
    )(s, v[:, 0, :], v[:, 1, :], v[:, 2, :], Ws_w.T, Ws_b.reshape(1, H), Wv_w.T)


def _sc_pos(posx, posy, posz, idx_i, idx_j, goff, ngroups):
    n = posx.shape[0]
    mesh = plsc.VectorSubcoreMesh(core_axis_name="c", subcore_axis_name="s")
    out3d = jax.ShapeDtypeStruct((ngroups, 8, _EG), jnp.float32)

    @functools.partial(
        pl.kernel,
        out_type=[out3d, out3d, out3d, out3d],
        mesh=mesh,
        compiler_params=pltpu.CompilerParams(needs_layout_passes=False),
        scratch_types=[
            pltpu.VMEM((n,), jnp.float32),
            pltpu.VMEM((n,), jnp.float32),
            pltpu.VMEM((n,), jnp.float32),
            pltpu.VMEM((_EG,), jnp.int32),
            pltpu.VMEM((_EG,), jnp.int32),
            pltpu.VMEM((8, _EG), jnp.float32),
            pltpu.VMEM((8, _EG), jnp.float32),
            pltpu.VMEM((8, _EG), jnp.float32),
            pltpu.VMEM((8, _EG), jnp.float32),
        ],
    )
    def k(px_hbm, py_hbm, pz_hbm, ii_hbm, jj_hbm,
          rx_hbm, ry_hbm, rz_hbm, d2_hbm,
          px_v, py_v, pz_v, ii_v, jj_v, sx, sy, sz, sd):
        c = lax.axis_index("c")
        s = lax.axis_index("s")
        w = c * _NS + s
        pltpu.sync_copy(px_hbm, px_v)
        pltpu.sync_copy(py_hbm, py_v)
        pltpu.sync_copy(pz_hbm, pz_v)
        nt = (ngroups - w + _NW - 1) // _NW

        def body(t, carry):
            g = w + t * _NW
            off = pl.multiple_of((goff + g) * _EG, _EG)
            pltpu.sync_copy(ii_hbm.at[pl.ds(off, _EG)], ii_v)
            pltpu.sync_copy(jj_hbm.at[pl.ds(off, _EG)], jj_v)
            for sub in range(8):
                i16 = ii_v[pl.ds(sub * 16, 16)]
                j16 = jj_v[pl.ds(sub * 16, 16)]
                rx = plsc.load_gather(px_v, [j16]) - plsc.load_gather(px_v, [i16])
                ry = plsc.load_gather(py_v, [j16]) - plsc.load_gather(py_v, [i16])
                rz = plsc.load_gather(pz_v, [j16]) - plsc.load_gather(pz_v, [i16])
                d2 = rx * rx + ry * ry + rz * rz
                for r in range(8):
                    sx[r, pl.ds(sub * 16, 16)] = rx
                    sy[r, pl.ds(sub * 16, 16)] = ry
                    sz[r, pl.ds(sub * 16, 16)] = rz
                    sd[r, pl.ds(sub * 16, 16)] = d2
            pltpu.sync_copy(sx, rx_hbm.at[g])
            pltpu.sync_copy(sy, ry_hbm.at[g])
            pltpu.sync_copy(sz, rz_hbm.at[g])
            pltpu.sync_copy(sd, d2_hbm.at[g])
            return carry

        lax.fori_loop(0, nt, body, 0)

    return k(posx, posy, posz, idx_i, idx_j)


def _sc_gather(t0, t1, idx_j, goff, ngroups):
    e = ngroups * _EG
    mesh = plsc.VectorSubcoreMesh(core_axis_name="c", subcore_axis_name="s")
    out2d = jax.ShapeDtypeStruct((e, H), jnp.float32)

    @functools.partial(
        pl.kernel,
        out_type=[out2d, out2d],
        mesh=mesh,
        compiler_params=pltpu.CompilerParams(needs_layout_passes=False),
        scratch_types=[
            pltpu.VMEM((_EG,), jnp.int32),
            pltpu.VMEM((_EG,), jnp.int32),
            pltpu.VMEM((_EG, H), jnp.float32),
            pltpu.VMEM((_EG, H), jnp.float32),
            pltpu.VMEM((_EG, H), jnp.float32),
            pltpu.VMEM((_EG, H), jnp.float32),
            pltpu.SemaphoreType.DMA,
            pltpu.SemaphoreType.DMA,
            pltpu.SemaphoreType.DMA,
            pltpu.SemaphoreType.DMA,
            pltpu.SemaphoreType.DMA,
            pltpu.SemaphoreType.DMA,
        ],
    )
    def k(t0_hbm, t1_hbm, jj_hbm, o0_hbm, o1_hbm,
          jj_a, jj_b, b0a, b1a, b0b, b1b,
          sia, sib, sga, sgb, swa, swb):
        c = lax.axis_index("c")
        s = lax.axis_index("s")
        w = c * _NS + s
        nt = (ngroups - w + _NW - 1) // _NW

        def _off(t):
            return pl.multiple_of((w + t * _NW) * _EG, _EG)

        def _ioff(t):
            return pl.multiple_of((goff + w + t * _NW) * _EG, _EG)

        def start_idx(t, jv, sem):
            pltpu.async_copy(jj_hbm.at[pl.ds(_ioff(t), _EG)], jv, sem)

        def wait_idx(t, jv, sem):
            pltpu.make_async_copy(jj_hbm.at[pl.ds(_ioff(t), _EG)], jv,
                                  sem).wait()

        def start_gath(jv, u0, u1, sem):
            pltpu.async_copy(t0_hbm.at[jv], u0, sem)
            pltpu.async_copy(t1_hbm.at[jv], u1, sem)

        def wait_gath(jv, u0, u1, sem):
            pltpu.make_async_copy(t0_hbm.at[jv], u0, sem).wait()
            pltpu.make_async_copy(t1_hbm.at[jv], u1, sem).wait()

        def start_wr(t, u0, u1, sem):
            off = _off(t)
            pltpu.async_copy(u0, o0_hbm.at[pl.ds(off, _EG), :], sem)
            pltpu.async_copy(u1, o1_hbm.at[pl.ds(off, _EG), :], sem)

        def wait_wr(t, u0, u1, sem):
            off = _off(t)
            pltpu.make_async_copy(u0, o0_hbm.at[pl.ds(off, _EG), :],
                                  sem).wait()
            pltpu.make_async_copy(u1, o1_hbm.at[pl.ds(off, _EG), :],
                                  sem).wait()

        start_idx(0, jj_a, sia)

        def pair(u, carry):
            t_a = 2 * u
            t_b = 2 * u + 1
            wait_idx(t_a, jj_a, sia)

            @pl.when(t_a > 0)
            def _():
                wait_wr(t_a - 2, b0a, b1a, swa)

            start_gath(jj_a, b0a, b1a, sga)

            @pl.when(t_b < nt)
            def _():
                start_idx(t_b, jj_b, sib)

            wait_gath(jj_a, b0a, b1a, sga)
            start_wr(t_a, b0a, b1a, swa)

            @pl.when(t_b < nt)
            def _():
                wait_idx(t_b, jj_b, sib)

                @pl.when(t_b > 1)
                def _():
                    wait_wr(t_b - 2, b0b, b1b, swb)

                start_gath(jj_b, b0b, b1b, sgb)

                @pl.when(t_b + 1 < nt)
                def _():
                    start_idx(t_b + 1, jj_a, sia)

                wait_gath(jj_b, b0b, b1b, sgb)
                start_wr(t_b, b0b, b1b, swb)

            return carry

        lax.fori_loop(0, (nt + 1) // 2, pair, 0)
        last_a = ((nt - 1) // 2) * 2
        wait_wr(last_a, b0a, b1a, swa)
        last_b = ((nt - 2) // 2) * 2 + 1

        @pl.when(last_b >= 0)
        def _():
            wait_wr(last_b, b0b, b1b, swb)

    return k(t0, t1, idx_j)


_BB = 10
_BE = _BB * _EG


def _k3_body(p0_ref, p1_ref, rx_ref, ry_ref, rz_ref, d2_ref,
             phi1pT_ref, b1c_ref, phi2_ref, b2c_ref, Wvb_ref,
             ms_ref, m0_ref, m1_ref, m2_ref):
    d2 = d2_ref[...]
    dist = jnp.sqrt(d2)
    invu = 1.0 / (dist + 1e-09)
    cv = 0.5 * (jnp.cos((jnp.pi / CUTOFF) * dist) + 1.0) \
        * (dist < CUTOFF).astype(jnp.float32)
    scl = cv / dist
    dist4 = jnp.concatenate([dist, dist, dist, dist], axis=1)
    scl4 = jnp.concatenate([scl, scl, scl, scl], axis=1)
    kidx = jax.lax.broadcasted_iota(jnp.int32, (1, 32, 1), 1)
    freq = jnp.where(kidx < N_RBF,
                     (kidx.astype(jnp.float32) + 1.0) * (jnp.pi / CUTOFF),
                     0.0)
    rbfT = jnp.sin(dist4 * freq) * scl4

    phi1pT = phi1pT_ref[...]
    phi2 = phi2_ref[...]
    b1c = b1c_ref[...]
    b2c = b2c_ref[...]
    w_rows, ux_rows, uy_rows, uz_rows = [], [], [], []
    for g in range(_BB):
        h1 = jnp.dot(phi1pT, rbfT[g],
                     preferred_element_type=jnp.float32) + b1c
        h = h1 * jax.nn.sigmoid(h1)
        wT = jnp.dot(phi2, h, preferred_element_type=jnp.float32) + b2c
        w_rows.append(wT.T)
        iv = invu[g]
        ux_rows.append((rx_ref[g] * iv).T[:, 0:1])
        uy_rows.append((ry_ref[g] * iv).T[:, 0:1])
        uz_rows.append((rz_ref[g] * iv).T[:, 0:1])
    w = jnp.concatenate(w_rows, axis=0)
    ux = jnp.concatenate(ux_rows, axis=0)
    uy = jnp.concatenate(uy_rows, axis=0)
    uz = jnp.concatenate(uz_rows, axis=0)

    pk0 = p0_ref[...]
    pk1 = p1_ref[...]
    ms_ref[...] = _unpack_lo(pk0) * w
    proj = (ux * _unpack_hi(pk0)
            + uy * _unpack_lo(pk1)
            + uz * _unpack_hi(pk1))
    qv = (proj + Wvb_ref[...]) * w
    m0_ref[...] = ux * qv
    m1_ref[...] = uy * qv
    m2_ref[...] = uz * qv


def _edge_compute(p0, p1, r8x, r8y, r8z, d28, cblk,
                  phi_w1, phi_b1, phi_w2, phi_b2, Wv_b):
    e = p0.shape[0]
    grid = e // _BE
    phi1pT = jnp.zeros((H, 32), jnp.float32).at[:, :N_RBF].set(phi_w1)
    ebs = pl.BlockSpec((_BE, H), lambda i: (i, 0))
    rbs = pl.BlockSpec((_BB, 8, _EG), lambda i: (i + cblk, 0, 0))
    return pl.pallas_call(
        _k3_body,
        grid=(grid,),
        in_specs=[
            ebs, ebs, rbs, rbs, rbs, rbs,
            pl.BlockSpec((H, 32), lambda i: (0, 0)),
            pl.BlockSpec((H, 1), lambda i: (0, 0)),
            pl.BlockSpec((H, H), lambda i: (0, 0)),
            pl.BlockSpec((H, 1), lambda i: (0, 0)),
            pl.BlockSpec((1, H), lambda i: (0, 0)),
        ],
        out_specs=[ebs, ebs, ebs, ebs],
        out_shape=[jax.ShapeDtypeStruct((e, H), jnp.float32)] * 4,
    )(p0, p1, r8x, r8y, r8z, d28,
      phi1pT, phi_b1.reshape(H, 1), phi_w2, phi_b2.reshape(H, 1),
      Wv_b.reshape(1, H))


def _sc_scatter(ms, mv0, mv1, mv2, idx_i, goff, n):
    e = ms.shape[0]
    ngroups = e // _EG
    rows0 = 624
    tail = n - rows0 * _NS
    mesh = plsc.VectorSubcoreMesh(core_axis_name="c", subcore_axis_name="s")
    outn = jax.ShapeDtypeStruct((n, H), jnp.float32)

    @functools.partial(
        pl.kernel,
        out_type=[outn, outn, outn, outn],
        mesh=mesh,
        compiler_params=pltpu.CompilerParams(needs_layout_passes=False),
        scratch_types=[
            pltpu.VMEM((_EG,), jnp.int32),
            pltpu.VMEM((_EG,), jnp.int32),
            pltpu.VMEM((_EG, H), jnp.float32),
            pltpu.VMEM((_EG, H), jnp.float32),
            pltpu.VMEM((16, H), jnp.float32),
            pltpu.VMEM_SHARED((n, H), jnp.float32),
            pltpu.SemaphoreType.DMA,
            pltpu.SemaphoreType.DMA,
        ],
    )
    def k(d0_hbm, d1_hbm, d2_hbm, d3_hbm, ii_hbm,
          o0_hbm, o1_hbm, o2_hbm, o3_hbm,
          ii_a, ii_b, buf_a, buf_b, zbuf, acc, sem_a, sem_b):
        c = lax.axis_index("c")
        s = lax.axis_index("s")

        def zfill(t, carry):
            r = t // 8
            kk = (t % 8) * 16
            zbuf[r, pl.ds(kk, 16)] = jnp.zeros((16,), jnp.float32)
            return carry

        lax.fori_loop(0, 16 * 8, zfill, 0)

        def do_comp(dat_hbm, out_hbm):
            base = pl.multiple_of(s * rows0, 8)

            def zero_body(t, carry):
                zoff = pl.multiple_of(base + t * 16, 8)
                pltpu.sync_copy(zbuf, acc.at[pl.ds(zoff, 16), :])
                return carry

            lax.fori_loop(0, rows0 // 16, zero_body, 0)

            @pl.when(s == _NS - 1)
            def _():
                pltpu.sync_copy(zbuf, acc.at[pl.ds(rows0 * _NS, tail), :])

            plsc.subcore_barrier()
            nt = (ngroups - s + _NS - 1) // _NS

            def _off(t):
                return pl.multiple_of((s + t * _NS) * _EG, _EG)

            def _ioff(t):
                return pl.multiple_of((goff + s + t * _NS) * _EG, _EG)

            def start_loads(t, iv, bv, sem):
                pltpu.async_copy(ii_hbm.at[pl.ds(_ioff(t), _EG)], iv, sem)
                pltpu.async_copy(dat_hbm.at[pl.ds(_off(t), _EG), :], bv, sem)

            def wait_loads(t, iv, bv, sem):
                pltpu.make_async_copy(ii_hbm.at[pl.ds(_ioff(t), _EG)], iv,
                                      sem).wait()
                pltpu.make_async_copy(dat_hbm.at[pl.ds(_off(t), _EG), :], bv,
                                      sem).wait()

            start_loads(0, ii_a, buf_a, sem_a)

            def pair(u, carry):
                t_a = 2 * u
                t_b = 2 * u + 1
                wait_loads(t_a, ii_a, buf_a, sem_a)

                @pl.when(t_b < nt)
                def _():
                    start_loads(t_b, ii_b, buf_b, sem_b)

                pltpu.sync_copy(buf_a, acc.at[ii_a], add=True)

                @pl.when(t_b < nt)
                def _():
                    wait_loads(t_b, ii_b, buf_b, sem_b)

                    @pl.when(t_b + 1 < nt)
                    def _():
                        start_loads(t_b + 1, ii_a, buf_a, sem_a)

                    pltpu.sync_copy(buf_b, acc.at[ii_b], add=True)

                return carry

            lax.fori_loop(0, (nt + 1) // 2, pair, 0)
            plsc.subcore_barrier()
            pltpu.sync_copy(acc.at[pl.ds(base, rows0), :],
                            out_hbm.at[pl.ds(base, rows0), :])

            @pl.when(s == _NS - 1)
            def _():
                pltpu.sync_copy(acc.at[pl.ds(rows0 * _NS, tail), :],
                                out_hbm.at[pl.ds(rows0 * _NS, tail), :])

        pairs = ((d0_hbm, o0_hbm), (d1_hbm, o1_hbm),
                 (d2_hbm, o2_hbm), (d3_hbm, o3_hbm))
        for ci, (dat, out) in enumerate(pairs):
            @pl.when(c == ci // 2)
            def _():
                do_comp(dat, out)

    return k(ms, mv0, mv1, mv2, idx_i)


def _k5_body(s_ref, v_ref, msA_ref, m0A_ref, m1A_ref, m2A_ref,
             msB_ref, m0B_ref, m1B_ref, m2B_ref,
             Us1T_ref, b1_ref, Us2T_ref, b2_ref, sout_ref, vout_ref):
    ms = msA_ref[...] + msB_ref[...]
    h1 = jnp.dot(ms, Us1T_ref[...],
                 preferred_element_type=jnp.float32) + b1_ref[...]
    h = h1 * jax.nn.sigmoid(h1)
    ds = jnp.dot(h, Us2T_ref[...], preferred_element_type=jnp.float32) + b2_ref[...]
    sout_ref[...] = s_ref[...] + ds
    vout_ref[:, 0, :] = v_ref[:, 0, :] + m0A_ref[...] + m0B_ref[...]
    vout_ref[:, 1, :] = v_ref[:, 1, :] + m1A_ref[...] + m1B_ref[...]
    vout_ref[:, 2, :] = v_ref[:, 2, :] + m2A_ref[...] + m2B_ref[...]


def _node_update(s, v, accA, accB, Us_w1, Us_b1, Us_w2, Us_b2,
                 block_n=2000):
    n = s.shape[0]
    grid = n // block_n
    bs = pl.BlockSpec((block_n, H), lambda i: (i, 0))
    vs = pl.BlockSpec((block_n, 3, H), lambda i: (i, 0, 0))
    ws = pl.BlockSpec((H, H), lambda i: (0, 0))
    cs = pl.BlockSpec((1, H), lambda i: (0, 0))
    return pl.pallas_call(
        _k5_body,
        grid=(grid,),
        in_specs=[bs, vs, bs, bs, bs, bs, bs, bs, bs, bs, ws, cs, ws, cs],
        out_specs=[bs, vs],
        out_shape=[
            jax.ShapeDtypeStruct((n, H), jnp.float32),
            jax.ShapeDtypeStruct((n, 3, H), jnp.float32),
        ],
    )(s, v, *accA, *accB, Us_w1.T, Us_b1.reshape(1, H),
      Us_w2.T, Us_b2.reshape(1, H))


def kernel(s, v, pos, edge_index, phi_w1, phi_b1, phi_w2, phi_b2, Ws_w, Ws_b,
           Wv_w, Wv_b, Us_w1, Us_b1, Us_w2, Us_b2):
    n = s.shape[0]
    i = edge_index[0]
    j = edge_index[1]
    posx = pos[:, 0]
    posy = pos[:, 1]
    posz = pos[:, 2]

    t0, t1 = _node_tables(s, v, Ws_w, Ws_b, Wv_w)

    e = i.shape[0]
    ngroups = e // _EG
    gh = ngroups // 2
    accs = []
    for ci in range(2):
        goff = ci * gh
        r8x, r8y, r8z, d28 = _sc_pos(posx, posy, posz, i, j, goff, gh)
        p0, p1 = _sc_gather(t0, t1, j, goff, gh)
        msgs = _edge_compute(
            p0, p1, r8x, r8y, r8z, d28, 0,
            phi_w1, phi_b1, phi_w2, phi_b2, Wv_b)
        accs.append(_sc_scatter(*msgs, i, goff, n))
    s_out, v_out = _node_update(s, v, accs[0], accs[1],
                                Us_w1, Us_b1, Us_w2, Us_b2)
    return (s_out, v_out)
